# initial kernel scaffold (unmeasured)
import jax
import jax.numpy as jnp
from jax import lax
from jax.experimental import pallas as pl
from jax.experimental.pallas import tpu as pltpu

N_DEV = 16
B, SQ, SKV, D = 2, 128, 128, 512
HQ_LOCAL, HKV_LOCAL, DH = 8, 2, 64
GROUP = 4
STEPS = 4


def kernel(x, Wq, Wo, K_ext, V_ext):
    def body(x_ref, wq_ref, wo_ref, k_ref, v_ref, out_ref,
             send_buf, recv_buf, send_sems, recv_sems):
        my = lax.axis_index("i")
        kv0 = my * HKV_LOCAL

        wq = wq_ref[:, :].astype(jnp.bfloat16)
        wo = wo_ref[:, :].astype(jnp.bfloat16)

        outs = []
        for b in range(B):
            xb = x_ref[b].astype(jnp.bfloat16)
            qb = jnp.dot(xb, wq, preferred_element_type=jnp.float32)
            heads = []
            for g in range(HKV_LOCAL):
                kg = k_ref[b, :, pl.ds(kv0 + g, 1), :].reshape(SKV, DH)
                vg = v_ref[b, :, pl.ds(kv0 + g, 1), :].reshape(SKV, DH)
                kg = kg.astype(jnp.bfloat16)
                vg = vg.astype(jnp.bfloat16)
                for hh in range(GROUP):
                    h = g * GROUP + hh
                    qh = qb[:, h * DH:(h + 1) * DH].astype(jnp.bfloat16)
                    s = jnp.dot(qh, kg.T,
                                preferred_element_type=jnp.float32) * 0.125
                    m = jnp.max(s, axis=-1, keepdims=True)
                    p = jnp.exp(s - m)
                    l = jnp.sum(p, axis=-1, keepdims=True)
                    o = jnp.dot(p.astype(jnp.bfloat16), vg,
                                preferred_element_type=jnp.float32) / l
                    heads.append(o)
            attn = jnp.concatenate(heads, axis=-1).astype(jnp.bfloat16)
            outs.append(jnp.dot(attn, wo, preferred_element_type=jnp.float32))
        acc = jnp.stack(outs)

        for step in range(STEPS):
            partner = my ^ (1 << step)
            send_buf[:, :, :] = acc
            rdma = pltpu.make_async_remote_copy(
                src_ref=send_buf,
                dst_ref=recv_buf.at[step],
                send_sem=send_sems.at[step],
                recv_sem=recv_sems.at[step],
                device_id=(partner,),
                device_id_type=pl.DeviceIdType.MESH,
            )
            rdma.start()
            rdma.wait()
            acc = acc + recv_buf[step]

        out_ref[:, :, :] = acc

    return pl.pallas_call(
        body,
        out_shape=jax.ShapeDtypeStruct((B, SQ, D), jnp.float32),
        in_specs=[pl.BlockSpec(memory_space=pltpu.VMEM)] * 5,
        out_specs=pl.BlockSpec(memory_space=pltpu.VMEM),
        scratch_shapes=[
            pltpu.VMEM((B, SQ, D), jnp.float32),
            pltpu.VMEM((STEPS, B, SQ, D), jnp.float32),
            pltpu.SemaphoreType.DMA((STEPS,)),
            pltpu.SemaphoreType.DMA((STEPS,)),
        ],
        compiler_params=pltpu.CompilerParams(collective_id=0),
    )(x, Wq, Wo, K_ext, V_ext)


# baseline (device time: 60853 ns/iter reference)
import jax
import jax.numpy as jnp
from jax import lax
from jax.experimental import pallas as pl
from jax.experimental.pallas import tpu as pltpu

N_DEV = 16
B, SQ, SKV, D = 2, 128, 128, 512
HQ_LOCAL, HKV_LOCAL, DH = 8, 2, 64
GROUP = 4
STEPS = 4


def kernel(x, Wq, Wo, K_ext, V_ext):
    def body(x_ref, wq_ref, wo_ref, k_ref, v_ref, out_ref,
             send_buf, recv_buf, send_sems, recv_sems):
        my = lax.axis_index("i")
        kv0 = my * HKV_LOCAL

        wq = wq_ref[:, :].astype(jnp.bfloat16)
        wo = wo_ref[:, :].astype(jnp.bfloat16)

        outs = []
        for b in range(B):
            xb = x_ref[b].astype(jnp.bfloat16)
            qb = jnp.dot(xb, wq, preferred_element_type=jnp.float32)
            heads = []
            for g in range(HKV_LOCAL):
                kg = k_ref[b, :, pl.ds(kv0 + g, 1), :].reshape(SKV, DH)
                vg = v_ref[b, :, pl.ds(kv0 + g, 1), :].reshape(SKV, DH)
                kg = kg.astype(jnp.bfloat16)
                vg = vg.astype(jnp.bfloat16)
                for hh in range(GROUP):
                    h = g * GROUP + hh
                    qh = qb[:, h * DH:(h + 1) * DH].astype(jnp.bfloat16)
                    s = jnp.dot(qh, kg.T,
                                preferred_element_type=jnp.float32) * 0.125
                    m = jnp.max(s, axis=-1, keepdims=True)
                    p = jnp.exp(s - m)
                    l = jnp.sum(p, axis=-1, keepdims=True)
                    o = jnp.dot(p.astype(jnp.bfloat16), vg,
                                preferred_element_type=jnp.float32) / l
                    heads.append(o)
            attn = jnp.concatenate(heads, axis=-1).astype(jnp.bfloat16)
            outs.append(jnp.dot(attn, wo, preferred_element_type=jnp.float32))
        acc = jnp.stack(outs)

        for step in range(STEPS):
            partner = my ^ (1 << step)
            send_buf[:, :, :] = acc
            rdma = pltpu.make_async_remote_copy(
                src_ref=send_buf,
                dst_ref=recv_buf.at[step],
                send_sem=send_sems.at[step],
                recv_sem=recv_sems.at[step],
                device_id=(partner,),
                device_id_type=pl.DeviceIdType.MESH,
            )
            rdma.start()
            rdma.wait()
            acc = acc + recv_buf[step]

        out_ref[:, :, :] = acc

    return pl.pallas_call(
        body,
        out_shape=jax.ShapeDtypeStruct((B, SQ, D), jnp.float32),
        in_specs=[pl.BlockSpec(memory_space=pltpu.VMEM)] * 5,
        out_specs=pl.BlockSpec(memory_space=pltpu.VMEM),
        scratch_shapes=[
            pltpu.VMEM((B, SQ, D), jnp.float32),
            pltpu.VMEM((STEPS, B, SQ, D), jnp.float32),
            pltpu.SemaphoreType.DMA((STEPS,)),
            pltpu.SemaphoreType.DMA((STEPS,)),
        ],
    )(x, Wq, Wo, K_ext, V_ext)


# device time: 37206 ns/iter; 1.6356x vs baseline; 1.6356x over previous
import jax
import jax.numpy as jnp
from jax import lax
from jax.experimental import pallas as pl
from jax.experimental.pallas import tpu as pltpu

N_DEV = 16
B, SQ, SKV, D = 2, 128, 128, 512
HQ_LOCAL, HKV_LOCAL, DH = 8, 2, 64
GROUP = 4
STEPS = 4


def kernel(x, Wq, Wo, K_ext, V_ext):
    def body(x_ref, wq_ref, wo_ref, k_ref, v_ref, out_ref,
             send_buf, recv_buf, send_sems, recv_sems):
        my = lax.axis_index("i")
        kv0 = my * HKV_LOCAL

        barrier_sem = pltpu.get_barrier_semaphore()
        partners = [my ^ (1 << s) for s in range(STEPS)]
        for p in partners:
            pl.semaphore_signal(barrier_sem, inc=1, device_id=(p,),
                                device_id_type=pl.DeviceIdType.MESH)

        wq = wq_ref[:, :].astype(jnp.bfloat16)
        wo = wo_ref[:, :].astype(jnp.bfloat16)

        outs = []
        for b in range(B):
            xb = x_ref[b].astype(jnp.bfloat16)
            qb = jnp.dot(xb, wq, preferred_element_type=jnp.float32)
            heads = []
            for g in range(HKV_LOCAL):
                kg = k_ref[b, :, pl.ds(kv0 + g, 1), :].reshape(SKV, DH)
                vg = v_ref[b, :, pl.ds(kv0 + g, 1), :].reshape(SKV, DH)
                kg = kg.astype(jnp.bfloat16)
                vg = vg.astype(jnp.bfloat16)
                qg = jnp.concatenate(
                    [qb[:, (g * GROUP + hh) * DH:(g * GROUP + hh + 1) * DH]
                     for hh in range(GROUP)], axis=0).astype(jnp.bfloat16)
                s = jnp.dot(qg, kg.T,
                            preferred_element_type=jnp.float32) * 0.125
                m = jnp.max(s, axis=-1, keepdims=True)
                p = jnp.exp(s - m)
                l = jnp.sum(p, axis=-1, keepdims=True)
                o = jnp.dot(p.astype(jnp.bfloat16), vg,
                            preferred_element_type=jnp.float32) / l
                for hh in range(GROUP):
                    heads.append(o[hh * SQ:(hh + 1) * SQ, :])
            attn = jnp.concatenate(heads, axis=-1).astype(jnp.bfloat16)
            outs.append(jnp.dot(attn, wo, preferred_element_type=jnp.float32))
        acc = jnp.stack(outs)

        pl.semaphore_wait(barrier_sem, STEPS)

        for step in range(STEPS):
            send_buf[:, :, :] = acc.astype(jnp.bfloat16)
            rdma = pltpu.make_async_remote_copy(
                src_ref=send_buf,
                dst_ref=recv_buf.at[step],
                send_sem=send_sems.at[step],
                recv_sem=recv_sems.at[step],
                device_id=(partners[step],),
                device_id_type=pl.DeviceIdType.MESH,
            )
            rdma.start()
            rdma.wait()
            acc = acc + recv_buf[step].astype(jnp.float32)

        out_ref[:, :, :] = acc

    return pl.pallas_call(
        body,
        out_shape=jax.ShapeDtypeStruct((B, SQ, D), jnp.float32),
        in_specs=[pl.BlockSpec(memory_space=pltpu.VMEM)] * 5,
        out_specs=pl.BlockSpec(memory_space=pltpu.VMEM),
        scratch_shapes=[
            pltpu.VMEM((B, SQ, D), jnp.bfloat16),
            pltpu.VMEM((STEPS, B, SQ, D), jnp.bfloat16),
            pltpu.SemaphoreType.DMA((STEPS,)),
            pltpu.SemaphoreType.DMA((STEPS,)),
        ],
        compiler_params=pltpu.CompilerParams(collective_id=0),
    )(x, Wq, Wo, K_ext, V_ext)


# device time: 13387 ns/iter; 4.5457x vs baseline; 2.7793x over previous
import jax
import jax.numpy as jnp
from jax import lax
from jax.experimental import pallas as pl
from jax.experimental.pallas import tpu as pltpu

N_DEV = 16
B, SQ, SKV, D = 2, 128, 128, 512
HQ_LOCAL, HKV_LOCAL, DH = 8, 2, 64
GROUP = 4
ROWS = SQ // N_DEV


def kernel(x, Wq, Wo, K_ext, V_ext):
    def body(x_ref, wq_ref, wo_ref, k_ref, v_ref, out_ref,
             kv_buf, part_bf, rs_buf, red_bf, ag_buf,
             kv_sems, rs_send_sems, rs_recv_sems, ag_send_sems,
             ag_recv_sems):
        my = lax.axis_index("i")
        kv0 = my * HKV_LOCAL

        kv_copies = []
        for t, ref in enumerate((k_ref, v_ref)):
            for b in range(B):
                c = pltpu.make_async_copy(
                    ref.at[b, :, pl.ds(kv0, HKV_LOCAL), :],
                    kv_buf.at[t, b],
                    kv_sems.at[t * B + b],
                )
                c.start()
                kv_copies.append(c)

        barrier_sem = pltpu.get_barrier_semaphore()
        for k in range(1, N_DEV):
            pl.semaphore_signal(barrier_sem, inc=1,
                                device_id=((my + k) % N_DEV,),
                                device_id_type=pl.DeviceIdType.MESH)

        wq = wq_ref[:, :].astype(jnp.bfloat16)
        wo = wo_ref[:, :].astype(jnp.bfloat16)
        for c in kv_copies:
            c.wait()

        def attention_partial(b):
            xb = x_ref[b].astype(jnp.bfloat16)
            qb = jnp.dot(xb, wq, preferred_element_type=jnp.float32)
            heads = []
            for g in range(HKV_LOCAL):
                kg = kv_buf[0, b, :, g, :].astype(jnp.bfloat16)
                vg = kv_buf[1, b, :, g, :].astype(jnp.bfloat16)
                qg = jnp.concatenate(
                    [qb[:, (g * GROUP + hh) * DH:(g * GROUP + hh + 1) * DH]
                     for hh in range(GROUP)], axis=0).astype(jnp.bfloat16)
                s = jnp.dot(qg, kg.T,
                            preferred_element_type=jnp.float32) * 0.125
                m = jnp.max(s, axis=-1, keepdims=True)
                p = jnp.exp(s - m)
                l = jnp.sum(p, axis=-1, keepdims=True)
                o = jnp.dot(p.astype(jnp.bfloat16), vg,
                            preferred_element_type=jnp.float32) / l
                for hh in range(GROUP):
                    heads.append(o[hh * SQ:(hh + 1) * SQ, :])
            attn = jnp.concatenate(heads, axis=-1).astype(jnp.bfloat16)
            return jnp.dot(attn, wo, preferred_element_type=jnp.float32)

        for b in range(B):
            part_bf[:, b, :] = attention_partial(b).astype(jnp.bfloat16)
        rs_buf[0] = part_bf[pl.ds(my * ROWS, ROWS)]

        pl.semaphore_wait(barrier_sem, N_DEV - 1)
        rs_rdmas = []
        for k in range(1, N_DEV):
            dest = (my + k) % N_DEV
            r = pltpu.make_async_remote_copy(
                src_ref=part_bf.at[pl.ds(dest * ROWS, ROWS)],
                dst_ref=rs_buf.at[N_DEV - k],
                send_sem=rs_send_sems.at[k - 1],
                recv_sem=rs_recv_sems.at[N_DEV - k - 1],
                device_id=(dest,),
                device_id_type=pl.DeviceIdType.MESH,
            )
            r.start()
            rs_rdmas.append(r)

        for r in rs_rdmas:
            r.wait_recv()
        red = jnp.sum(rs_buf[:].astype(jnp.float32), axis=0)
        red_bf[:, :, :] = red.astype(jnp.bfloat16)
        for b in range(B):
            out_ref[b, pl.ds(my * ROWS, ROWS), :] = red[:, b, :]

        ag_rdmas = []
        for k in range(1, N_DEV):
            dest = (my + k) % N_DEV
            r = pltpu.make_async_remote_copy(
                src_ref=red_bf,
                dst_ref=ag_buf.at[N_DEV - k],
                send_sem=ag_send_sems.at[k - 1],
                recv_sem=ag_recv_sems.at[N_DEV - k - 1],
                device_id=(dest,),
                device_id_type=pl.DeviceIdType.MESH,
            )
            r.start()
            ag_rdmas.append(r)

        for j in range(1, N_DEV):
            ag_rdmas[N_DEV - 1 - j].wait_recv()
            src = (my + j) % N_DEV
            piece = ag_buf[j].astype(jnp.float32)
            for b in range(B):
                out_ref[b, pl.ds(src * ROWS, ROWS), :] = piece[:, b, :]

        for r in rs_rdmas:
            r.wait_send()
        for r in ag_rdmas:
            r.wait_send()

    return pl.pallas_call(
        body,
        out_shape=jax.ShapeDtypeStruct((B, SQ, D), jnp.float32),
        in_specs=(
            [pl.BlockSpec(memory_space=pltpu.VMEM)] * 3
            + [pl.BlockSpec(memory_space=pltpu.MemorySpace.HBM)] * 2
        ),
        out_specs=pl.BlockSpec(memory_space=pltpu.VMEM),
        scratch_shapes=[
            pltpu.VMEM((2, B, SKV, HKV_LOCAL, DH), jnp.float32),
            pltpu.VMEM((SQ, B, D), jnp.bfloat16),
            pltpu.VMEM((N_DEV, ROWS, B, D), jnp.bfloat16),
            pltpu.VMEM((ROWS, B, D), jnp.bfloat16),
            pltpu.VMEM((N_DEV, ROWS, B, D), jnp.bfloat16),
            pltpu.SemaphoreType.DMA((2 * B,)),
            pltpu.SemaphoreType.DMA((N_DEV - 1,)),
            pltpu.SemaphoreType.DMA((N_DEV - 1,)),
            pltpu.SemaphoreType.DMA((N_DEV - 1,)),
            pltpu.SemaphoreType.DMA((N_DEV - 1,)),
        ],
        compiler_params=pltpu.CompilerParams(collective_id=0),
    )(x, Wq, Wo, K_ext, V_ext)
